# Initial kernel scaffold; baseline (speedup 1.0000x reference)
import jax
import jax.numpy as jnp
from jax.experimental import pallas as pl


def kernel(hidden_states, router_logits, w13_weight, w2_weight):
    raise NotImplementedError("write your pallas kernel here")


# speedup vs baseline: 1.0871x; 1.0871x over previous
import functools

import jax
import jax.numpy as jnp
from jax import lax
from jax.experimental import pallas as pl
from jax.experimental.pallas import tpu as pltpu

NUM_EXPERTS = 8
TOP_K = 2
HIDDEN = 1024
INTER = 2048
TOKENS = 2048

TC_CHUNK = 256


def _moe_body(x_ref, logits_ref, w13_ref, w2_ref, out_ref, acc_ref):
    e = pl.program_id(0)
    tc = pl.program_id(1)

    logits = logits_ref[...].astype(jnp.float32)
    m = jnp.max(logits, axis=1, keepdims=True)
    p = jnp.exp(logits - m)
    p = p / jnp.sum(p, axis=1, keepdims=True)
    iota = lax.broadcasted_iota(jnp.int32, p.shape, 1)
    w0 = jnp.max(p, axis=1)
    i0 = jnp.min(jnp.where(p == w0[:, None], iota, NUM_EXPERTS), axis=1)
    p1 = jnp.where(iota == i0[:, None], -1.0, p)
    w1 = jnp.max(p1, axis=1)
    i1 = jnp.min(jnp.where(p1 == w1[:, None], iota, NUM_EXPERTS), axis=1)
    denom = w0 + w1
    we = (jnp.where(i0 == e, w0, 0.0) + jnp.where(i1 == e, w1, 0.0)) / denom

    x = x_ref[...].astype(jnp.bfloat16)
    w13 = w13_ref[0]
    w2 = w2_ref[0]
    h = lax.dot_general(x, w13, (((1,), (1,)), ((), ())),
                        preferred_element_type=jnp.float32)
    gate = h[:, :INTER]
    up = h[:, INTER:]
    act = (gate * jax.nn.sigmoid(gate) * up).astype(jnp.bfloat16)
    y = lax.dot_general(act, w2, (((1,), (1,)), ((), ())),
                        preferred_element_type=jnp.float32)
    wy = y * we[:, None]

    rows = pl.ds(tc * TC_CHUNK, TC_CHUNK)

    @pl.when(e == 0)
    def _():
        acc_ref[rows, :] = wy

    @pl.when(e != 0)
    def _():
        acc_ref[rows, :] = acc_ref[rows, :] + wy

    out_ref[...] = acc_ref[rows, :]


@functools.partial(jax.jit, static_argnames=())
def kernel(hidden_states, router_logits, w13_weight, w2_weight):
    n_tc = TOKENS // TC_CHUNK
    grid = (NUM_EXPERTS, n_tc)
    w13_bf = w13_weight.astype(jnp.bfloat16)
    w2_bf = w2_weight.astype(jnp.bfloat16)
    return pl.pallas_call(
        _moe_body,
        grid=grid,
        in_specs=[
            pl.BlockSpec((TC_CHUNK, HIDDEN), lambda e, tc: (tc, 0)),
            pl.BlockSpec((TC_CHUNK, NUM_EXPERTS), lambda e, tc: (tc, 0)),
            pl.BlockSpec((1, 2 * INTER, HIDDEN), lambda e, tc: (e, 0, 0)),
            pl.BlockSpec((1, HIDDEN, INTER), lambda e, tc: (e, 0, 0)),
        ],
        out_specs=pl.BlockSpec((TC_CHUNK, HIDDEN), lambda e, tc: (tc, 0)),
        out_shape=jax.ShapeDtypeStruct((TOKENS, HIDDEN), jnp.float32),
        scratch_shapes=[pltpu.VMEM((TOKENS, HIDDEN), jnp.float32)],
        compiler_params=pltpu.CompilerParams(
            dimension_semantics=("arbitrary", "arbitrary"),
        ),
    )(hidden_states, router_logits, w13_bf, w2_bf)
